# baseline (speedup 1.0000x reference)
import functools

import jax
import jax.numpy as jnp
from jax import lax
from jax.experimental import pallas as pl
from jax.experimental.pallas import tpu as pltpu
from jax.experimental.pallas import tpu_sc as plsc

_KC = 2048


def _ids_kernel(x2_ref, cb_ref, xn_ref, cn_ref, ids_ref):
    x2 = x2_ref[...]
    cb = cb_ref[...]
    xn = xn_ref[...]
    cn = cn_ref[...]
    bb = x2.shape[0]
    k = cb.shape[0]

    acc_v = jnp.full((bb, 1), jnp.inf, jnp.float32)
    acc_i = jnp.zeros((bb, 1), jnp.int32)
    iota_f = lax.broadcasted_iota(
        jnp.int32, (bb, _KC), 1).astype(jnp.float32)
    for c0 in range(0, k, _KC):
        xc2 = lax.dot_general(
            x2, cb[c0:c0 + _KC, :], (((1,), (1,)), ((), ())),
            preferred_element_type=jnp.float32)
        dist = (xn + cn[:, c0:c0 + _KC]) - xc2
        mv = jnp.min(dist, axis=1, keepdims=True)
        mi_f = jnp.min(jnp.where(dist == mv, iota_f, float(_KC)),
                       axis=1, keepdims=True)
        mi = c0 + mi_f.astype(jnp.int32)
        keep = (acc_v < mv) | jnp.isnan(acc_v) | ((acc_v == mv) & (acc_i < mi))
        acc_v = jnp.where(keep, acc_v, mv)
        acc_i = jnp.where(keep, acc_i, mi)
        acc_v = acc_v.astype(jnp.bfloat16).astype(jnp.float32)
    ids_ref[...] = acc_i[:, 0]


def _compute_ids(x, codebook):
    b, d = x.shape
    k = codebook.shape[0]
    bb = 4096
    x2 = (2.0 * x).astype(jnp.bfloat16)
    cbb = codebook.astype(jnp.bfloat16)
    xn = (x ** 2).sum(axis=1, keepdims=True)
    cn = (codebook.T ** 2).sum(axis=0, keepdims=True)

    return pl.pallas_call(
        _ids_kernel,
        grid=(b // bb,),
        in_specs=[
            pl.BlockSpec((bb, d), lambda i: (i, 0)),
            pl.BlockSpec((k, d), lambda i: (0, 0)),
            pl.BlockSpec((bb, 1), lambda i: (i, 0)),
            pl.BlockSpec((1, k), lambda i: (0, 0)),
        ],
        out_specs=pl.BlockSpec((bb,), lambda i: (i,)),
        out_shape=jax.ShapeDtypeStruct((b,), jnp.int32),
        compiler_params=pltpu.CompilerParams(
            dimension_semantics=("parallel",)),
    )(x2, cbb, xn, cn)


def _make_gather(b, d, dp):
    info = plsc.get_sparse_core_info()
    nw = info.num_cores * info.num_subcores
    b_per_w = b // nw
    mesh = plsc.VectorSubcoreMesh(core_axis_name="c", subcore_axis_name="s")

    @functools.partial(
        pl.kernel, mesh=mesh,
        out_type=jax.ShapeDtypeStruct((b, dp), jnp.float32),
        scratch_types=[
            pltpu.VMEM((b_per_w,), jnp.int32),
            pltpu.VMEM((b_per_w, dp), jnp.float32),
            pltpu.SemaphoreType.DMA,
        ],
    )
    def gather(table_hbm, idx_hbm, out_hbm, idx_v, rows_v, sem):
        wid = lax.axis_index("s") * info.num_cores + lax.axis_index("c")
        base = wid * b_per_w
        pltpu.sync_copy(idx_hbm.at[pl.ds(base, b_per_w)], idx_v)
        pltpu.async_copy(table_hbm.at[idx_v], rows_v, sem).wait()
        pltpu.sync_copy(rows_v, out_hbm.at[pl.ds(base, b_per_w)])

    return gather


def kernel(x, temperature, codebook):
    del temperature
    b, d = x.shape
    dp = 128
    cb_pad = jnp.pad(codebook, ((0, 0), (0, dp - d)))
    ids = _compute_ids(x, codebook)
    emb = _make_gather(b, d, dp)(cb_pad, ids)[:, :d]
    return emb, ids


# speedup vs baseline: 1.1929x; 1.1929x over previous
import functools

import jax
import jax.numpy as jnp
from jax import lax
from jax.experimental import pallas as pl
from jax.experimental.pallas import tpu as pltpu
from jax.experimental.pallas import tpu_sc as plsc

_KC = 2048


def _ids_kernel(x2_ref, cb_ref, xn_ref, cn_ref, ids_ref):
    x2 = x2_ref[...]
    cb = cb_ref[...]
    xn = xn_ref[...]
    cn = cn_ref[...]
    bb = x2.shape[0]
    k = cb.shape[0]

    acc_v = jnp.full((bb, 1), jnp.inf, jnp.float32)
    acc_i = jnp.zeros((bb, 1), jnp.int32)
    iota_f = lax.broadcasted_iota(
        jnp.int32, (bb, _KC), 1).astype(jnp.float32)
    for c0 in range(0, k, _KC):
        xc2 = lax.dot_general(
            x2, cb[c0:c0 + _KC, :], (((1,), (1,)), ((), ())),
            preferred_element_type=jnp.float32)
        dist = (xn + cn[:, c0:c0 + _KC]) - xc2
        mv = jnp.min(dist, axis=1, keepdims=True)
        mi = c0 + jnp.argmin(dist, axis=1)[:, None]
        keep = (acc_v < mv) | jnp.isnan(acc_v) | ((acc_v == mv) & (acc_i < mi))
        acc_v = jnp.where(keep, acc_v, mv)
        acc_i = jnp.where(keep, acc_i, mi)
        acc_v = acc_v.astype(jnp.bfloat16).astype(jnp.float32)
    ids_ref[...] = acc_i[:, 0]


def _compute_ids(x, codebook):
    b, d = x.shape
    k = codebook.shape[0]
    bb = 2048
    x2 = (2.0 * x).astype(jnp.bfloat16)
    cbb = codebook.astype(jnp.bfloat16)
    xn = (x ** 2).sum(axis=1, keepdims=True)
    cn = (codebook.T ** 2).sum(axis=0, keepdims=True)

    return pl.pallas_call(
        _ids_kernel,
        grid=(b // bb,),
        in_specs=[
            pl.BlockSpec((bb, d), lambda i: (i, 0)),
            pl.BlockSpec((k, d), lambda i: (0, 0)),
            pl.BlockSpec((bb, 1), lambda i: (i, 0)),
            pl.BlockSpec((1, k), lambda i: (0, 0)),
        ],
        out_specs=pl.BlockSpec((bb,), lambda i: (i,)),
        out_shape=jax.ShapeDtypeStruct((b,), jnp.int32),
        compiler_params=pltpu.CompilerParams(
            dimension_semantics=("parallel",)),
    )(x2, cbb, xn, cn)


def _make_gather(b, d, dp):
    info = plsc.get_sparse_core_info()
    nw = info.num_cores * info.num_subcores
    b_per_w = b // nw
    mesh = plsc.VectorSubcoreMesh(core_axis_name="c", subcore_axis_name="s")

    @functools.partial(
        pl.kernel, mesh=mesh,
        out_type=jax.ShapeDtypeStruct((b, dp), jnp.float32),
        scratch_types=[
            pltpu.VMEM((b_per_w,), jnp.int32),
            pltpu.VMEM((b_per_w, dp), jnp.float32),
            pltpu.SemaphoreType.DMA,
        ],
    )
    def gather(table_hbm, idx_hbm, out_hbm, idx_v, rows_v, sem):
        wid = lax.axis_index("s") * info.num_cores + lax.axis_index("c")
        base = wid * b_per_w
        pltpu.sync_copy(idx_hbm.at[pl.ds(base, b_per_w)], idx_v)
        pltpu.async_copy(table_hbm.at[idx_v], rows_v, sem).wait()
        pltpu.sync_copy(rows_v, out_hbm.at[pl.ds(base, b_per_w)])

    return gather


def kernel(x, temperature, codebook):
    del temperature
    b, d = x.shape
    dp = 128
    cb_pad = jnp.pad(codebook, ((0, 0), (0, dp - d)))
    ids = _compute_ids(x, codebook)
    emb = _make_gather(b, d, dp)(cb_pad, ids)[:, :d]
    return emb, ids


# speedup vs baseline: 1.2867x; 1.0787x over previous
import functools

import jax
import jax.numpy as jnp
from jax import lax
from jax.experimental import pallas as pl
from jax.experimental.pallas import tpu as pltpu
from jax.experimental.pallas import tpu_sc as plsc

_KC = 2048


def _ids_kernel(x2_ref, cb_ref, xn_ref, cn_ref, ids_ref):
    x2 = x2_ref[...]
    cb = cb_ref[...]
    xn = xn_ref[...]
    cn = cn_ref[...]
    bb = x2.shape[0]
    k = cb.shape[0]

    acc_v = jnp.full((bb, 1), jnp.inf, jnp.float32)
    acc_i = jnp.zeros((bb, 1), jnp.int32)
    iota_f = lax.broadcasted_iota(
        jnp.int32, (1, _KC), 1).astype(jnp.float32)
    for c0 in range(0, k, _KC):
        xc2 = lax.dot_general(
            x2, cb[c0:c0 + _KC, :], (((1,), (1,)), ((), ())),
            preferred_element_type=jnp.float32)
        dist = (xn + cn[:, c0:c0 + _KC]) - xc2
        mv = jnp.min(dist, axis=1, keepdims=True)
        mi_f = jnp.min(jnp.where(dist == mv, iota_f, float(_KC)),
                       axis=1, keepdims=True)
        mi = c0 + mi_f.astype(jnp.int32)
        keep = (acc_v < mv) | jnp.isnan(acc_v) | ((acc_v == mv) & (acc_i < mi))
        acc_v = jnp.where(keep, acc_v, mv)
        acc_i = jnp.where(keep, acc_i, mi)
        acc_v = acc_v.astype(jnp.bfloat16).astype(jnp.float32)
    ids_ref[...] = acc_i[:, 0]


def _compute_ids(x, codebook):
    b, d = x.shape
    k = codebook.shape[0]
    bb = 2048
    x2 = (2.0 * x).astype(jnp.bfloat16)
    cbb = codebook.astype(jnp.bfloat16)
    xn = (x ** 2).sum(axis=1, keepdims=True)
    cn = (codebook.T ** 2).sum(axis=0, keepdims=True)

    return pl.pallas_call(
        _ids_kernel,
        grid=(b // bb,),
        in_specs=[
            pl.BlockSpec((bb, d), lambda i: (i, 0)),
            pl.BlockSpec((k, d), lambda i: (0, 0)),
            pl.BlockSpec((bb, 1), lambda i: (i, 0)),
            pl.BlockSpec((1, k), lambda i: (0, 0)),
        ],
        out_specs=pl.BlockSpec((bb,), lambda i: (i,)),
        out_shape=jax.ShapeDtypeStruct((b,), jnp.int32),
        compiler_params=pltpu.CompilerParams(
            dimension_semantics=("parallel",)),
    )(x2, cbb, xn, cn)


def _make_gather(b, d, dp):
    info = plsc.get_sparse_core_info()
    nw = info.num_cores * info.num_subcores
    b_per_w = b // nw
    mesh = plsc.VectorSubcoreMesh(core_axis_name="c", subcore_axis_name="s")

    @functools.partial(
        pl.kernel, mesh=mesh,
        out_type=jax.ShapeDtypeStruct((b, dp), jnp.float32),
        scratch_types=[
            pltpu.VMEM((b_per_w,), jnp.int32),
            pltpu.VMEM((b_per_w, dp), jnp.float32),
            pltpu.SemaphoreType.DMA,
        ],
    )
    def gather(table_hbm, idx_hbm, out_hbm, idx_v, rows_v, sem):
        wid = lax.axis_index("s") * info.num_cores + lax.axis_index("c")
        base = wid * b_per_w
        pltpu.sync_copy(idx_hbm.at[pl.ds(base, b_per_w)], idx_v)
        pltpu.async_copy(table_hbm.at[idx_v], rows_v, sem).wait()
        pltpu.sync_copy(rows_v, out_hbm.at[pl.ds(base, b_per_w)])

    return gather


def kernel(x, temperature, codebook):
    del temperature
    b, d = x.shape
    dp = 128
    cb_pad = jnp.pad(codebook, ((0, 0), (0, dp - d)))
    ids = _compute_ids(x, codebook)
    emb = _make_gather(b, d, dp)(cb_pad, ids)[:, :d]
    return emb, ids


# speedup vs baseline: 1.3242x; 1.0291x over previous
import functools

import jax
import jax.numpy as jnp
from jax import lax
from jax.experimental import pallas as pl
from jax.experimental.pallas import tpu as pltpu
from jax.experimental.pallas import tpu_sc as plsc

_KC = 2048


def _ids_kernel(x_ref, cb_ref, xn_ref, cn_ref, ids_ref):
    x_blk = x_ref[...]
    x2 = (x_blk + x_blk).astype(jnp.bfloat16)
    cb = cb_ref[...].astype(jnp.bfloat16)
    xn = xn_ref[...]
    cn = cn_ref[...]
    bb = x2.shape[0]
    k = cb.shape[0]

    acc_v = jnp.full((bb, 1), jnp.inf, jnp.float32)
    acc_i = jnp.zeros((bb, 1), jnp.int32)
    iota_f = lax.broadcasted_iota(
        jnp.int32, (1, _KC), 1).astype(jnp.float32)
    for c0 in range(0, k, _KC):
        xc2 = lax.dot_general(
            x2, cb[c0:c0 + _KC, :], (((1,), (1,)), ((), ())),
            preferred_element_type=jnp.float32)
        dist = (xn + cn[:, c0:c0 + _KC]) - xc2
        mv = jnp.min(dist, axis=1, keepdims=True)
        mi_f = jnp.min(jnp.where(dist == mv, iota_f, float(_KC)),
                       axis=1, keepdims=True)
        mi = c0 + mi_f.astype(jnp.int32)
        keep = (acc_v < mv) | jnp.isnan(acc_v) | ((acc_v == mv) & (acc_i < mi))
        acc_v = jnp.where(keep, acc_v, mv)
        acc_i = jnp.where(keep, acc_i, mi)
        acc_v = acc_v.astype(jnp.bfloat16).astype(jnp.float32)
    ids_ref[...] = acc_i[:, 0]


def _compute_ids(x, codebook):
    b, d = x.shape
    k = codebook.shape[0]
    bb = 2048
    xn = (x ** 2).sum(axis=1, keepdims=True)
    cn = (codebook.T ** 2).sum(axis=0, keepdims=True)

    return pl.pallas_call(
        _ids_kernel,
        grid=(b // bb,),
        in_specs=[
            pl.BlockSpec((bb, d), lambda i: (i, 0)),
            pl.BlockSpec((k, d), lambda i: (0, 0)),
            pl.BlockSpec((bb, 1), lambda i: (i, 0)),
            pl.BlockSpec((1, k), lambda i: (0, 0)),
        ],
        out_specs=pl.BlockSpec((bb,), lambda i: (i,)),
        out_shape=jax.ShapeDtypeStruct((b,), jnp.int32),
        compiler_params=pltpu.CompilerParams(
            dimension_semantics=("parallel",)),
    )(x, codebook, xn, cn)


def _make_gather(b, d, dp):
    info = plsc.get_sparse_core_info()
    nw = info.num_cores * info.num_subcores
    b_per_w = b // nw
    mesh = plsc.VectorSubcoreMesh(core_axis_name="c", subcore_axis_name="s")

    @functools.partial(
        pl.kernel, mesh=mesh,
        out_type=jax.ShapeDtypeStruct((b, dp), jnp.float32),
        scratch_types=[
            pltpu.VMEM((b_per_w,), jnp.int32),
            pltpu.VMEM((b_per_w, dp), jnp.float32),
            pltpu.SemaphoreType.DMA,
        ],
    )
    def gather(table_hbm, idx_hbm, out_hbm, idx_v, rows_v, sem):
        wid = lax.axis_index("s") * info.num_cores + lax.axis_index("c")
        base = wid * b_per_w
        pltpu.sync_copy(idx_hbm.at[pl.ds(base, b_per_w)], idx_v)
        pltpu.async_copy(table_hbm.at[idx_v], rows_v, sem).wait()
        pltpu.sync_copy(rows_v, out_hbm.at[pl.ds(base, b_per_w)])

    return gather


def kernel(x, temperature, codebook):
    del temperature
    b, d = x.shape
    dp = 128
    cb_pad = jnp.pad(codebook, ((0, 0), (0, dp - d)))
    ids = _compute_ids(x, codebook)
    emb = _make_gather(b, d, dp)(cb_pad, ids)[:, :d]
    return emb, ids
